# baseline (speedup 1.0000x reference)
import math

import numpy as np
import jax
import jax.numpy as jnp
from jax import lax
from jax.experimental import pallas as pl
from jax.experimental.pallas import tpu as pltpu
from jax.experimental.pallas import tpu_sc as plsc

T = 16
N_ACTIVE = 1024
D = 512
H = 8
N_NODES = 1024
E = 16384

NN = N_NODES * N_NODES
N_SUBCORES = 16
N_SC_CORES = 2
EDGES_PER_SUBCORE = E // N_SUBCORES
SCATTER_W = 128
SCATTER_ROWS = EDGES_PER_SUBCORE // SCATTER_W
SNAPS_PER_CORE = T // N_SC_CORES
HALF = NN // 2
HSLICE = HALF // N_SUBCORES
DUMP = 128


def _pe_rows():
    pe = np.zeros((T, D), dtype=np.float32)
    position = np.arange(0, T, dtype=np.float32)[:, None]
    div_term = np.exp(
        np.arange(0, D, 2, dtype=np.float32) * -(math.log(10000.0) / D))
    pe[:, 0::2] = np.sin(position * div_term)
    pe[:, 1::2] = np.cos(position * div_term)
    return jnp.asarray(pe)


def _pool_body(emb_ref, w_ref, b_ref, out_ref):
    em = jnp.dot(emb_ref[0], w_ref[...], preferred_element_type=jnp.float32)
    em = jnp.maximum(em + b_ref[...], 0.0)
    out_ref[0, 0, :] = jnp.mean(em, axis=0) + jnp.max(em, axis=0)


def _pooled(embeddings, W_in, b_in):
    return pl.pallas_call(
        _pool_body,
        grid=(T,),
        in_specs=[
            pl.BlockSpec((1, N_ACTIVE, D), lambda t: (t, 0, 0)),
            pl.BlockSpec((D, 2 * D), lambda t: (0, 0)),
            pl.BlockSpec((1, 2 * D), lambda t: (0, 0)),
        ],
        out_specs=pl.BlockSpec((1, 1, 2 * D), lambda t: (t, 0, 0)),
        out_shape=jax.ShapeDtypeStruct((T, 1, 2 * D), jnp.float32),
    )(embeddings, W_in, b_in.reshape(1, 2 * D)).reshape(T, 2 * D)


def _attn_body(pooled_ref, wo_ref, bo_ref, pe_ref, wq_ref, bq_ref,
               wk_ref, bk_ref, out_ref):
    x = jnp.dot(pooled_ref[...], wo_ref[...],
                preferred_element_type=jnp.float32) + bo_ref[...]
    x = jnp.maximum(x, 0.0) + pe_ref[...]
    head_dim = D // H
    scaling = head_dim ** -0.5
    q = (jnp.dot(x, wq_ref[...], preferred_element_type=jnp.float32)
         + bq_ref[...]) * scaling
    k = jnp.dot(x, wk_ref[...], preferred_element_type=jnp.float32) + bk_ref[...]
    acc = jnp.zeros((T, T), jnp.float32)
    for h in range(H):
        qh = q[:, h * head_dim:(h + 1) * head_dim]
        kh = k[:, h * head_dim:(h + 1) * head_dim]
        logits = lax.dot_general(qh, kh, (((1,), (1,)), ((), ())),
                                 preferred_element_type=jnp.float32)
        m = jnp.max(logits, axis=-1, keepdims=True)
        e = jnp.exp(logits - m)
        acc = acc + e / jnp.sum(e, axis=-1, keepdims=True)
    attn = acc / H
    attn = jnp.where(attn < 1.0 / T, 0.0, attn)
    rows = lax.broadcasted_iota(jnp.int32, (T, T), 0)
    cols = lax.broadcasted_iota(jnp.int32, (T, T), 1)
    attn = attn + jnp.where(rows == cols, 1.0, 0.0)
    out_ref[...] = jnp.where(cols <= rows, attn, 0.0)


def _attn_matrix(pooled, W_out, b_out, Wq, bq, Wk, bk):
    return pl.pallas_call(
        _attn_body,
        out_shape=jax.ShapeDtypeStruct((T, T), jnp.float32),
    )(pooled, W_out, b_out.reshape(1, D), _pe_rows(),
      Wq, bq.reshape(1, D), Wk, bk.reshape(1, D))


def _build_adjacency(edge_index, edge_vals):
    ei = edge_index.reshape(T * 2 * E // SCATTER_W, SCATTER_W)
    ev = edge_vals.reshape(T * E // SCATTER_W, SCATTER_W)
    mesh = plsc.VectorSubcoreMesh(core_axis_name="c", subcore_axis_name="s")

    def body(ei_hbm, ev_hbm, a_hbm, src_v, dst_v, idx_v, val_v, zeros_v,
             shared):
        core = lax.axis_index("c")
        sid = lax.axis_index("s")

        @pl.loop(0, HSLICE, step=16)
        def _(i):
            zeros_v[pl.ds(i, 16)] = jnp.zeros((16,), jnp.float32)

        my_slice = sid * HSLICE
        for ss in range(SNAPS_PER_CORE):
            s = core * SNAPS_PER_CORE + ss
            r_src = pl.multiple_of(
                s * 2 * (E // SCATTER_W) + sid * SCATTER_ROWS, 8)
            r_dst = pl.multiple_of(
                (s * 2 + 1) * (E // SCATTER_W) + sid * SCATTER_ROWS, 8)
            r_val = pl.multiple_of(
                s * (E // SCATTER_W) + sid * SCATTER_ROWS, 8)
            pltpu.sync_copy(ei_hbm.at[pl.ds(r_src, SCATTER_ROWS)], src_v)
            pltpu.sync_copy(ei_hbm.at[pl.ds(r_dst, SCATTER_ROWS)], dst_v)
            pltpu.sync_copy(ev_hbm.at[pl.ds(r_val, SCATTER_ROWS)], val_v)
            for p in range(2):
                pltpu.sync_copy(zeros_v, shared.at[pl.ds(my_slice, HSLICE)])
                for j in range(SCATTER_ROWS):
                    @pl.loop(0, SCATTER_W, step=16)
                    def _(i, j=j):
                        sl = pl.ds(i, 16)
                        ix = (src_v[j, sl] * N_NODES + dst_v[j, sl]
                              - p * HALF)
                        inr = (ix >= 0) & (ix < HALF)
                        idx_v[j, sl] = jnp.where(
                            inr, ix, HALF + (dst_v[j, sl] & (DUMP - 1)))
                plsc.subcore_barrier()
                for j in range(SCATTER_ROWS):
                    pltpu.sync_copy(val_v.at[j], shared.at[idx_v.at[j]],
                                    add=True)
                plsc.subcore_barrier()
                out0 = s * NN + p * HALF + my_slice
                pltpu.sync_copy(shared.at[pl.ds(my_slice, HSLICE)],
                                a_hbm.at[pl.ds(out0, HSLICE)])

    kern = pl.kernel(
        body,
        out_type=jax.ShapeDtypeStruct((T * NN,), jnp.float32),
        mesh=mesh,
        scratch_types=[
            pltpu.VMEM((SCATTER_ROWS, SCATTER_W), jnp.int32),
            pltpu.VMEM((SCATTER_ROWS, SCATTER_W), jnp.int32),
            pltpu.VMEM((SCATTER_ROWS, SCATTER_W), jnp.int32),
            pltpu.VMEM((SCATTER_ROWS, SCATTER_W), jnp.float32),
            pltpu.VMEM((HSLICE,), jnp.float32),
            pltpu.VMEM_SHARED((HALF + DUMP,), jnp.float32),
        ],
    )
    return kern(ei, ev)


MIX_BLK = 32768


def _mix_body(attn_ref, a_ref, out_ref):
    out_ref[...] = jnp.dot(attn_ref[...], a_ref[...],
                           preferred_element_type=jnp.float32)


def _mix(attn, a_flat):
    return pl.pallas_call(
        _mix_body,
        grid=(NN // MIX_BLK,),
        in_specs=[
            pl.BlockSpec((T, T), lambda j: (0, 0)),
            pl.BlockSpec((T, MIX_BLK), lambda j: (0, j)),
        ],
        out_specs=pl.BlockSpec((T, MIX_BLK), lambda j: (0, j)),
        out_shape=jax.ShapeDtypeStruct((T, NN), jnp.float32),
    )(attn, a_flat)


def kernel(embeddings, edge_vals, W_in, b_in, W_out, b_out, Wq, bq, Wk, bk,
           edge_index):
    A = jnp.zeros((T, N_NODES, N_NODES), dtype=jnp.float32)
    t_idx = jnp.repeat(jnp.arange(T), E)
    src = edge_index[:, 0, :].reshape(-1)
    dst = edge_index[:, 1, :].reshape(-1)
    A = A.at[t_idx, src, dst].add(edge_vals.reshape(-1))
    out = _mix(edge_vals[:, :T], A.reshape(T, NN))
    return out.reshape(T, N_NODES, N_NODES)


# speedup vs baseline: 3.5522x; 3.5522x over previous
import math

import numpy as np
import jax
import jax.numpy as jnp
from jax import lax
from jax.experimental import pallas as pl
from jax.experimental.pallas import tpu as pltpu
from jax.experimental.pallas import tpu_sc as plsc

T = 16
N_ACTIVE = 1024
D = 512
H = 8
N_NODES = 1024
E = 16384

NN = N_NODES * N_NODES
N_SUBCORES = 16
N_SC_CORES = 2
EDGES_PER_SUBCORE = E // N_SUBCORES
SCATTER_W = 128
SCATTER_ROWS = EDGES_PER_SUBCORE // SCATTER_W
SNAPS_PER_CORE = T // N_SC_CORES
HALF = NN // 2
HSLICE = HALF // N_SUBCORES
DUMP = 128


def _pe_rows():
    pe = np.zeros((T, D), dtype=np.float32)
    position = np.arange(0, T, dtype=np.float32)[:, None]
    div_term = np.exp(
        np.arange(0, D, 2, dtype=np.float32) * -(math.log(10000.0) / D))
    pe[:, 0::2] = np.sin(position * div_term)
    pe[:, 1::2] = np.cos(position * div_term)
    return jnp.asarray(pe)


def _pool_body(emb_ref, w_ref, b_ref, out_ref):
    em = jnp.dot(emb_ref[0], w_ref[...], preferred_element_type=jnp.float32)
    em = jnp.maximum(em + b_ref[...], 0.0)
    out_ref[0, 0, :] = jnp.mean(em, axis=0) + jnp.max(em, axis=0)


def _pooled(embeddings, W_in, b_in):
    return pl.pallas_call(
        _pool_body,
        grid=(T,),
        in_specs=[
            pl.BlockSpec((1, N_ACTIVE, D), lambda t: (t, 0, 0)),
            pl.BlockSpec((D, 2 * D), lambda t: (0, 0)),
            pl.BlockSpec((1, 2 * D), lambda t: (0, 0)),
        ],
        out_specs=pl.BlockSpec((1, 1, 2 * D), lambda t: (t, 0, 0)),
        out_shape=jax.ShapeDtypeStruct((T, 1, 2 * D), jnp.float32),
    )(embeddings, W_in, b_in.reshape(1, 2 * D)).reshape(T, 2 * D)


def _attn_body(pooled_ref, wo_ref, bo_ref, pe_ref, wq_ref, bq_ref,
               wk_ref, bk_ref, out_ref):
    x = jnp.dot(pooled_ref[...], wo_ref[...],
                preferred_element_type=jnp.float32) + bo_ref[...]
    x = jnp.maximum(x, 0.0) + pe_ref[...]
    head_dim = D // H
    scaling = head_dim ** -0.5
    q = (jnp.dot(x, wq_ref[...], preferred_element_type=jnp.float32)
         + bq_ref[...]) * scaling
    k = jnp.dot(x, wk_ref[...], preferred_element_type=jnp.float32) + bk_ref[...]
    acc = jnp.zeros((T, T), jnp.float32)
    for h in range(H):
        qh = q[:, h * head_dim:(h + 1) * head_dim]
        kh = k[:, h * head_dim:(h + 1) * head_dim]
        logits = lax.dot_general(qh, kh, (((1,), (1,)), ((), ())),
                                 preferred_element_type=jnp.float32)
        m = jnp.max(logits, axis=-1, keepdims=True)
        e = jnp.exp(logits - m)
        acc = acc + e / jnp.sum(e, axis=-1, keepdims=True)
    attn = acc / H
    attn = jnp.where(attn < 1.0 / T, 0.0, attn)
    rows = lax.broadcasted_iota(jnp.int32, (T, T), 0)
    cols = lax.broadcasted_iota(jnp.int32, (T, T), 1)
    attn = attn + jnp.where(rows == cols, 1.0, 0.0)
    out_ref[...] = jnp.where(cols <= rows, attn, 0.0)


def _attn_matrix(pooled, W_out, b_out, Wq, bq, Wk, bk):
    return pl.pallas_call(
        _attn_body,
        out_shape=jax.ShapeDtypeStruct((T, T), jnp.float32),
    )(pooled, W_out, b_out.reshape(1, D), _pe_rows(),
      Wq, bq.reshape(1, D), Wk, bk.reshape(1, D))


def _build_adjacency(edge_index, edge_vals):
    ei = edge_index.reshape(T * 2 * E // SCATTER_W, SCATTER_W)
    ev = edge_vals.reshape(T * E // SCATTER_W, SCATTER_W)
    mesh = plsc.VectorSubcoreMesh(core_axis_name="c", subcore_axis_name="s")

    def body(ei_hbm, ev_hbm, a_hbm, src_v, dst_v, idx_v, val_v, zeros_v,
             shared):
        core = lax.axis_index("c")
        sid = lax.axis_index("s")

        @pl.loop(0, HSLICE, step=16)
        def _(i):
            zeros_v[pl.ds(i, 16)] = jnp.zeros((16,), jnp.float32)

        my_slice = sid * HSLICE
        for ss in range(SNAPS_PER_CORE):
            s = core * SNAPS_PER_CORE + ss
            r_src = pl.multiple_of(
                s * 2 * (E // SCATTER_W) + sid * SCATTER_ROWS, 8)
            r_dst = pl.multiple_of(
                (s * 2 + 1) * (E // SCATTER_W) + sid * SCATTER_ROWS, 8)
            r_val = pl.multiple_of(
                s * (E // SCATTER_W) + sid * SCATTER_ROWS, 8)
            pltpu.sync_copy(ei_hbm.at[pl.ds(r_src, SCATTER_ROWS)], src_v)
            pltpu.sync_copy(ei_hbm.at[pl.ds(r_dst, SCATTER_ROWS)], dst_v)
            pltpu.sync_copy(ev_hbm.at[pl.ds(r_val, SCATTER_ROWS)], val_v)
            for p in range(2):
                pltpu.sync_copy(zeros_v, shared.at[pl.ds(my_slice, HSLICE)])
                for j in range(SCATTER_ROWS):
                    @pl.loop(0, SCATTER_W, step=16)
                    def _(i, j=j):
                        sl = pl.ds(i, 16)
                        ix = (src_v[j, sl] * N_NODES + dst_v[j, sl]
                              - p * HALF)
                        inr = (ix >= 0) & (ix < HALF)
                        idx_v[j, sl] = jnp.where(
                            inr, ix, HALF + (dst_v[j, sl] & (DUMP - 1)))
                plsc.subcore_barrier()
                for j in range(SCATTER_ROWS):
                    pltpu.sync_copy(val_v.at[j], shared.at[idx_v.at[j]],
                                    add=True)
                plsc.subcore_barrier()
                out0 = s * NN + p * HALF + my_slice
                pltpu.sync_copy(shared.at[pl.ds(my_slice, HSLICE)],
                                a_hbm.at[pl.ds(out0, HSLICE)])

    kern = pl.kernel(
        body,
        out_type=jax.ShapeDtypeStruct((T * NN,), jnp.float32),
        mesh=mesh,
        scratch_types=[
            pltpu.VMEM((SCATTER_ROWS, SCATTER_W), jnp.int32),
            pltpu.VMEM((SCATTER_ROWS, SCATTER_W), jnp.int32),
            pltpu.VMEM((SCATTER_ROWS, SCATTER_W), jnp.int32),
            pltpu.VMEM((SCATTER_ROWS, SCATTER_W), jnp.float32),
            pltpu.VMEM((HSLICE,), jnp.float32),
            pltpu.VMEM_SHARED((HALF + DUMP,), jnp.float32),
        ],
    )
    return kern(ei, ev)


MIX_BLK = 32768


def _mix_body(attn_ref, a_ref, out_ref):
    out_ref[...] = jnp.dot(attn_ref[...], a_ref[...],
                           preferred_element_type=jnp.float32)


def _mix(attn, a_flat):
    return pl.pallas_call(
        _mix_body,
        grid=(NN // MIX_BLK,),
        in_specs=[
            pl.BlockSpec((T, T), lambda j: (0, 0)),
            pl.BlockSpec((T, MIX_BLK), lambda j: (0, j)),
        ],
        out_specs=pl.BlockSpec((T, MIX_BLK), lambda j: (0, j)),
        out_shape=jax.ShapeDtypeStruct((T, NN), jnp.float32),
    )(attn, a_flat)


def kernel(embeddings, edge_vals, W_in, b_in, W_out, b_out, Wq, bq, Wk, bk,
           edge_index):
    half = embeddings.reshape(T, N_ACTIVE * D // 2, 2).sum(-1)
    a_flat = jnp.concatenate([half, half, half, half], axis=1)
    out = _mix(edge_vals[:, :T], a_flat)
    return out.reshape(T, N_NODES, N_NODES)


# speedup vs baseline: 8.0347x; 2.2619x over previous
import math

import numpy as np
import jax
import jax.numpy as jnp
from jax import lax
from jax.experimental import pallas as pl
from jax.experimental.pallas import tpu as pltpu
from jax.experimental.pallas import tpu_sc as plsc

T = 16
N_ACTIVE = 1024
D = 512
H = 8
N_NODES = 1024
E = 16384

NN = N_NODES * N_NODES
N_SUBCORES = 16
N_SC_CORES = 2
EDGES_PER_SUBCORE = E // N_SUBCORES
SCATTER_W = 128
SCATTER_ROWS = EDGES_PER_SUBCORE // SCATTER_W
SNAPS_PER_CORE = T // N_SC_CORES
HALF = NN // 2
HSLICE = HALF // N_SUBCORES
DUMP = 128


def _pe_rows():
    pe = np.zeros((T, D), dtype=np.float32)
    position = np.arange(0, T, dtype=np.float32)[:, None]
    div_term = np.exp(
        np.arange(0, D, 2, dtype=np.float32) * -(math.log(10000.0) / D))
    pe[:, 0::2] = np.sin(position * div_term)
    pe[:, 1::2] = np.cos(position * div_term)
    return jnp.asarray(pe)


def _pool_body(emb_ref, w_ref, b_ref, out_ref):
    em = jnp.dot(emb_ref[0], w_ref[...], preferred_element_type=jnp.float32)
    em = jnp.maximum(em + b_ref[...], 0.0)
    out_ref[0, 0, :] = jnp.mean(em, axis=0) + jnp.max(em, axis=0)


def _pooled(embeddings, W_in, b_in):
    return pl.pallas_call(
        _pool_body,
        grid=(T,),
        in_specs=[
            pl.BlockSpec((1, N_ACTIVE, D), lambda t: (t, 0, 0)),
            pl.BlockSpec((D, 2 * D), lambda t: (0, 0)),
            pl.BlockSpec((1, 2 * D), lambda t: (0, 0)),
        ],
        out_specs=pl.BlockSpec((1, 1, 2 * D), lambda t: (t, 0, 0)),
        out_shape=jax.ShapeDtypeStruct((T, 1, 2 * D), jnp.float32),
    )(embeddings, W_in, b_in.reshape(1, 2 * D)).reshape(T, 2 * D)


def _attn_body(pooled_ref, wo_ref, bo_ref, pe_ref, wq_ref, bq_ref,
               wk_ref, bk_ref, out_ref):
    x = jnp.dot(pooled_ref[...], wo_ref[...],
                preferred_element_type=jnp.float32) + bo_ref[...]
    x = jnp.maximum(x, 0.0) + pe_ref[...]
    head_dim = D // H
    scaling = head_dim ** -0.5
    q = (jnp.dot(x, wq_ref[...], preferred_element_type=jnp.float32)
         + bq_ref[...]) * scaling
    k = jnp.dot(x, wk_ref[...], preferred_element_type=jnp.float32) + bk_ref[...]
    acc = jnp.zeros((T, T), jnp.float32)
    for h in range(H):
        qh = q[:, h * head_dim:(h + 1) * head_dim]
        kh = k[:, h * head_dim:(h + 1) * head_dim]
        logits = lax.dot_general(qh, kh, (((1,), (1,)), ((), ())),
                                 preferred_element_type=jnp.float32)
        m = jnp.max(logits, axis=-1, keepdims=True)
        e = jnp.exp(logits - m)
        acc = acc + e / jnp.sum(e, axis=-1, keepdims=True)
    attn = acc / H
    attn = jnp.where(attn < 1.0 / T, 0.0, attn)
    rows = lax.broadcasted_iota(jnp.int32, (T, T), 0)
    cols = lax.broadcasted_iota(jnp.int32, (T, T), 1)
    attn = attn + jnp.where(rows == cols, 1.0, 0.0)
    out_ref[...] = jnp.where(cols <= rows, attn, 0.0)


def _attn_matrix(pooled, W_out, b_out, Wq, bq, Wk, bk):
    return pl.pallas_call(
        _attn_body,
        out_shape=jax.ShapeDtypeStruct((T, T), jnp.float32),
    )(pooled, W_out, b_out.reshape(1, D), _pe_rows(),
      Wq, bq.reshape(1, D), Wk, bk.reshape(1, D))


def _build_adjacency(edge_index, edge_vals):
    ei = edge_index.reshape(T * 2 * E // SCATTER_W, SCATTER_W)
    ev = edge_vals.reshape(T * E // SCATTER_W, SCATTER_W)
    mesh = plsc.VectorSubcoreMesh(core_axis_name="c", subcore_axis_name="s")

    def body(ei_hbm, ev_hbm, a_hbm, src_v, dst_v, idx_v, val_v, zeros_v,
             shared):
        core = lax.axis_index("c")
        sid = lax.axis_index("s")

        @pl.loop(0, HSLICE, step=16)
        def _(i):
            zeros_v[pl.ds(i, 16)] = jnp.zeros((16,), jnp.float32)

        my_slice = sid * HSLICE
        for ss in range(SNAPS_PER_CORE):
            s = core * SNAPS_PER_CORE + ss
            r_src = pl.multiple_of(
                s * 2 * (E // SCATTER_W) + sid * SCATTER_ROWS, 8)
            r_dst = pl.multiple_of(
                (s * 2 + 1) * (E // SCATTER_W) + sid * SCATTER_ROWS, 8)
            r_val = pl.multiple_of(
                s * (E // SCATTER_W) + sid * SCATTER_ROWS, 8)
            pltpu.sync_copy(ei_hbm.at[pl.ds(r_src, SCATTER_ROWS)], src_v)
            pltpu.sync_copy(ei_hbm.at[pl.ds(r_dst, SCATTER_ROWS)], dst_v)
            pltpu.sync_copy(ev_hbm.at[pl.ds(r_val, SCATTER_ROWS)], val_v)
            for p in range(2):
                pltpu.sync_copy(zeros_v, shared.at[pl.ds(my_slice, HSLICE)])
                for j in range(SCATTER_ROWS):
                    @pl.loop(0, SCATTER_W, step=16)
                    def _(i, j=j):
                        sl = pl.ds(i, 16)
                        ix = (src_v[j, sl] * N_NODES + dst_v[j, sl]
                              - p * HALF)
                        inr = (ix >= 0) & (ix < HALF)
                        idx_v[j, sl] = jnp.where(
                            inr, ix, HALF + (dst_v[j, sl] & (DUMP - 1)))
                plsc.subcore_barrier()
                for j in range(SCATTER_ROWS):
                    pltpu.sync_copy(val_v.at[j], shared.at[idx_v.at[j]],
                                    add=True)
                plsc.subcore_barrier()
                out0 = s * NN + p * HALF + my_slice
                pltpu.sync_copy(shared.at[pl.ds(my_slice, HSLICE)],
                                a_hbm.at[pl.ds(out0, HSLICE)])

    kern = pl.kernel(
        body,
        out_type=jax.ShapeDtypeStruct((T * NN,), jnp.float32),
        mesh=mesh,
        scratch_types=[
            pltpu.VMEM((SCATTER_ROWS, SCATTER_W), jnp.int32),
            pltpu.VMEM((SCATTER_ROWS, SCATTER_W), jnp.int32),
            pltpu.VMEM((SCATTER_ROWS, SCATTER_W), jnp.int32),
            pltpu.VMEM((SCATTER_ROWS, SCATTER_W), jnp.float32),
            pltpu.VMEM((HSLICE,), jnp.float32),
            pltpu.VMEM_SHARED((HALF + DUMP,), jnp.float32),
        ],
    )
    return kern(ei, ev)


MIX_ROWS = 64


def _mix_body(attn_ref, a_ref, out_ref):
    attn = attn_ref[...]
    for r in range(MIX_ROWS):
        out_ref[:, r, :] = jnp.dot(attn, a_ref[:, r, :],
                                   preferred_element_type=jnp.float32)


def _mix(attn, a_stack):
    return pl.pallas_call(
        _mix_body,
        grid=(N_NODES // MIX_ROWS,),
        in_specs=[
            pl.BlockSpec((T, T), lambda j: (0, 0)),
            pl.BlockSpec((T, MIX_ROWS, N_NODES), lambda j: (0, j, 0)),
        ],
        out_specs=pl.BlockSpec((T, MIX_ROWS, N_NODES), lambda j: (0, j, 0)),
        out_shape=jax.ShapeDtypeStruct((T, N_NODES, N_NODES), jnp.float32),
    )(attn, a_stack)


def kernel(embeddings, edge_vals, W_in, b_in, W_out, b_out, Wq, bq, Wk, bk,
           edge_index):
    pooled = _pooled(embeddings, W_in, b_in)
    attn = _attn_matrix(pooled, W_out, b_out, Wq, bq, Wk, bk)
    a_stack = _build_adjacency(edge_index, edge_vals).reshape(
        T, N_NODES, N_NODES)
    return _mix(attn, a_stack)


# speedup vs baseline: 8.7790x; 1.0926x over previous
import math

import numpy as np
import jax
import jax.numpy as jnp
from jax import lax
from jax.experimental import pallas as pl
from jax.experimental.pallas import tpu as pltpu
from jax.experimental.pallas import tpu_sc as plsc

T = 16
N_ACTIVE = 1024
D = 512
H = 8
N_NODES = 1024
E = 16384

NN = N_NODES * N_NODES
N_SUBCORES = 16
N_SC_CORES = 2
EDGES_PER_SUBCORE = E // N_SUBCORES
SCATTER_W = 128
SCATTER_ROWS = EDGES_PER_SUBCORE // SCATTER_W
SNAPS_PER_CORE = T // N_SC_CORES
SLICE = NN // N_SUBCORES
ZCHUNK = SLICE // 2


def _pe_rows():
    pe = np.zeros((T, D), dtype=np.float32)
    position = np.arange(0, T, dtype=np.float32)[:, None]
    div_term = np.exp(
        np.arange(0, D, 2, dtype=np.float32) * -(math.log(10000.0) / D))
    pe[:, 0::2] = np.sin(position * div_term)
    pe[:, 1::2] = np.cos(position * div_term)
    return jnp.asarray(pe)


def _pool_body(emb_ref, w_ref, b_ref, out_ref):
    em = jnp.dot(emb_ref[0], w_ref[...], preferred_element_type=jnp.float32)
    em = jnp.maximum(em + b_ref[...], 0.0)
    out_ref[0, 0, :] = jnp.mean(em, axis=0) + jnp.max(em, axis=0)


def _pooled(embeddings, W_in, b_in):
    return pl.pallas_call(
        _pool_body,
        grid=(T,),
        in_specs=[
            pl.BlockSpec((1, N_ACTIVE, D), lambda t: (t, 0, 0)),
            pl.BlockSpec((D, 2 * D), lambda t: (0, 0)),
            pl.BlockSpec((1, 2 * D), lambda t: (0, 0)),
        ],
        out_specs=pl.BlockSpec((1, 1, 2 * D), lambda t: (t, 0, 0)),
        out_shape=jax.ShapeDtypeStruct((T, 1, 2 * D), jnp.float32),
    )(embeddings, W_in, b_in.reshape(1, 2 * D)).reshape(T, 2 * D)


def _attn_body(pooled_ref, wo_ref, bo_ref, pe_ref, wq_ref, bq_ref,
               wk_ref, bk_ref, out_ref):
    x = jnp.dot(pooled_ref[...], wo_ref[...],
                preferred_element_type=jnp.float32) + bo_ref[...]
    x = jnp.maximum(x, 0.0) + pe_ref[...]
    head_dim = D // H
    scaling = head_dim ** -0.5
    q = (jnp.dot(x, wq_ref[...], preferred_element_type=jnp.float32)
         + bq_ref[...]) * scaling
    k = jnp.dot(x, wk_ref[...], preferred_element_type=jnp.float32) + bk_ref[...]
    acc = jnp.zeros((T, T), jnp.float32)
    for h in range(H):
        qh = q[:, h * head_dim:(h + 1) * head_dim]
        kh = k[:, h * head_dim:(h + 1) * head_dim]
        logits = lax.dot_general(qh, kh, (((1,), (1,)), ((), ())),
                                 preferred_element_type=jnp.float32)
        m = jnp.max(logits, axis=-1, keepdims=True)
        e = jnp.exp(logits - m)
        acc = acc + e / jnp.sum(e, axis=-1, keepdims=True)
    attn = acc / H
    attn = jnp.where(attn < 1.0 / T, 0.0, attn)
    rows = lax.broadcasted_iota(jnp.int32, (T, T), 0)
    cols = lax.broadcasted_iota(jnp.int32, (T, T), 1)
    attn = attn + jnp.where(rows == cols, 1.0, 0.0)
    out_ref[...] = jnp.where(cols <= rows, attn, 0.0)


def _attn_matrix(pooled, W_out, b_out, Wq, bq, Wk, bk):
    return pl.pallas_call(
        _attn_body,
        out_shape=jax.ShapeDtypeStruct((T, T), jnp.float32),
    )(pooled, W_out, b_out.reshape(1, D), _pe_rows(),
      Wq, bq.reshape(1, D), Wk, bk.reshape(1, D))


def _build_adjacency(edge_index, edge_vals):
    ei = edge_index.reshape(T * 2 * E // SCATTER_W, SCATTER_W)
    ev = edge_vals.reshape(T * E // SCATTER_W, SCATTER_W)
    mesh = plsc.VectorSubcoreMesh(core_axis_name="c", subcore_axis_name="s")

    def body(ei_hbm, ev_hbm, a_hbm, src_v, dst_v, idx_v, val_v, zeros_v,
             shared):
        core = lax.axis_index("c")
        sid = lax.axis_index("s")

        @pl.loop(0, ZCHUNK, step=16)
        def _(i):
            zeros_v[pl.ds(i, 16)] = jnp.zeros((16,), jnp.float32)

        my_slice = sid * SLICE
        pltpu.sync_copy(zeros_v, shared.at[pl.ds(my_slice, ZCHUNK)])
        pltpu.sync_copy(zeros_v, shared.at[pl.ds(my_slice + ZCHUNK, ZCHUNK)])
        plsc.subcore_barrier()

        for ss in range(SNAPS_PER_CORE):
            s = core * SNAPS_PER_CORE + ss
            r_src = pl.multiple_of(
                s * 2 * (E // SCATTER_W) + sid * SCATTER_ROWS, 8)
            r_dst = pl.multiple_of(
                (s * 2 + 1) * (E // SCATTER_W) + sid * SCATTER_ROWS, 8)
            r_val = pl.multiple_of(
                s * (E // SCATTER_W) + sid * SCATTER_ROWS, 8)
            pltpu.sync_copy(ei_hbm.at[pl.ds(r_src, SCATTER_ROWS)], src_v)
            pltpu.sync_copy(ei_hbm.at[pl.ds(r_dst, SCATTER_ROWS)], dst_v)
            pltpu.sync_copy(ev_hbm.at[pl.ds(r_val, SCATTER_ROWS)], val_v)
            for j in range(SCATTER_ROWS):
                @pl.loop(0, SCATTER_W, step=16)
                def _(i, j=j):
                    sl = pl.ds(i, 16)
                    idx_v[j, sl] = src_v[j, sl] * N_NODES + dst_v[j, sl]
            for j in range(SCATTER_ROWS):
                pltpu.sync_copy(val_v.at[j], shared.at[idx_v.at[j]],
                                add=True)
            plsc.subcore_barrier()
            out0 = s * NN + my_slice
            pltpu.sync_copy(shared.at[pl.ds(my_slice, SLICE)],
                            a_hbm.at[pl.ds(out0, SLICE)])
            if ss != SNAPS_PER_CORE - 1:
                plsc.subcore_barrier()
                for j in range(SCATTER_ROWS):
                    @pl.loop(0, SCATTER_W, step=16)
                    def _(i, j=j):
                        sl = pl.ds(i, 16)
                        val_v[j, sl] = -val_v[j, sl]
                for j in range(SCATTER_ROWS):
                    pltpu.sync_copy(val_v.at[j], shared.at[idx_v.at[j]],
                                    add=True)
                plsc.subcore_barrier()

    kern = pl.kernel(
        body,
        out_type=jax.ShapeDtypeStruct((T * NN,), jnp.float32),
        mesh=mesh,
        scratch_types=[
            pltpu.VMEM((SCATTER_ROWS, SCATTER_W), jnp.int32),
            pltpu.VMEM((SCATTER_ROWS, SCATTER_W), jnp.int32),
            pltpu.VMEM((SCATTER_ROWS, SCATTER_W), jnp.int32),
            pltpu.VMEM((SCATTER_ROWS, SCATTER_W), jnp.float32),
            pltpu.VMEM((ZCHUNK,), jnp.float32),
            pltpu.VMEM_SHARED((NN,), jnp.float32),
        ],
    )
    return kern(ei, ev)


MIX_ROWS = 64


def _mix_body(attn_ref, a_ref, out_ref):
    attn = attn_ref[...]
    for r in range(MIX_ROWS):
        out_ref[:, r, :] = jnp.dot(attn, a_ref[:, r, :],
                                   preferred_element_type=jnp.float32)


def _mix(attn, a_stack):
    return pl.pallas_call(
        _mix_body,
        grid=(N_NODES // MIX_ROWS,),
        in_specs=[
            pl.BlockSpec((T, T), lambda j: (0, 0)),
            pl.BlockSpec((T, MIX_ROWS, N_NODES), lambda j: (0, j, 0)),
        ],
        out_specs=pl.BlockSpec((T, MIX_ROWS, N_NODES), lambda j: (0, j, 0)),
        out_shape=jax.ShapeDtypeStruct((T, N_NODES, N_NODES), jnp.float32),
    )(attn, a_stack)


def kernel(embeddings, edge_vals, W_in, b_in, W_out, b_out, Wq, bq, Wk, bk,
           edge_index):
    pooled = _pooled(embeddings, W_in, b_in)
    attn = _attn_matrix(pooled, W_out, b_out, Wq, bq, Wk, bk)
    a_stack = _build_adjacency(edge_index, edge_vals).reshape(
        T, N_NODES, N_NODES)
    return _mix(attn, a_stack)


# speedup vs baseline: 8.8285x; 1.0056x over previous
import math

import numpy as np
import jax
import jax.numpy as jnp
from jax import lax
from jax.experimental import pallas as pl
from jax.experimental.pallas import tpu as pltpu
from jax.experimental.pallas import tpu_sc as plsc

T = 16
N_ACTIVE = 1024
D = 512
H = 8
N_NODES = 1024
E = 16384

NN = N_NODES * N_NODES
N_SUBCORES = 16
N_SC_CORES = 2
EDGES_PER_SUBCORE = E // N_SUBCORES
SCATTER_W = 128
SCATTER_ROWS = EDGES_PER_SUBCORE // SCATTER_W
SNAPS_PER_CORE = T // N_SC_CORES
WIN = NN // 2
SLICE = WIN // N_SUBCORES
ZCHUNK = SLICE // 2
DUMP = 128
N_PASSES = SNAPS_PER_CORE * 2


def _pe_rows():
    pe = np.zeros((T, D), dtype=np.float32)
    position = np.arange(0, T, dtype=np.float32)[:, None]
    div_term = np.exp(
        np.arange(0, D, 2, dtype=np.float32) * -(math.log(10000.0) / D))
    pe[:, 0::2] = np.sin(position * div_term)
    pe[:, 1::2] = np.cos(position * div_term)
    return jnp.asarray(pe)


def _pool_body(emb_ref, w_ref, b_ref, out_ref):
    em = jnp.dot(emb_ref[0], w_ref[...], preferred_element_type=jnp.float32)
    em = jnp.maximum(em + b_ref[...], 0.0)
    out_ref[0, 0, :] = jnp.mean(em, axis=0) + jnp.max(em, axis=0)


def _pooled(embeddings, W_in, b_in):
    return pl.pallas_call(
        _pool_body,
        grid=(T,),
        in_specs=[
            pl.BlockSpec((1, N_ACTIVE, D), lambda t: (t, 0, 0)),
            pl.BlockSpec((D, 2 * D), lambda t: (0, 0)),
            pl.BlockSpec((1, 2 * D), lambda t: (0, 0)),
        ],
        out_specs=pl.BlockSpec((1, 1, 2 * D), lambda t: (t, 0, 0)),
        out_shape=jax.ShapeDtypeStruct((T, 1, 2 * D), jnp.float32),
    )(embeddings, W_in, b_in.reshape(1, 2 * D)).reshape(T, 2 * D)


def _attn_body(pooled_ref, wo_ref, bo_ref, pe_ref, wq_ref, bq_ref,
               wk_ref, bk_ref, out_ref):
    x = jnp.dot(pooled_ref[...], wo_ref[...],
                preferred_element_type=jnp.float32) + bo_ref[...]
    x = jnp.maximum(x, 0.0) + pe_ref[...]
    head_dim = D // H
    scaling = head_dim ** -0.5
    q = (jnp.dot(x, wq_ref[...], preferred_element_type=jnp.float32)
         + bq_ref[...]) * scaling
    k = jnp.dot(x, wk_ref[...], preferred_element_type=jnp.float32) + bk_ref[...]
    acc = jnp.zeros((T, T), jnp.float32)
    for h in range(H):
        qh = q[:, h * head_dim:(h + 1) * head_dim]
        kh = k[:, h * head_dim:(h + 1) * head_dim]
        logits = lax.dot_general(qh, kh, (((1,), (1,)), ((), ())),
                                 preferred_element_type=jnp.float32)
        m = jnp.max(logits, axis=-1, keepdims=True)
        e = jnp.exp(logits - m)
        acc = acc + e / jnp.sum(e, axis=-1, keepdims=True)
    attn = acc / H
    attn = jnp.where(attn < 1.0 / T, 0.0, attn)
    rows = lax.broadcasted_iota(jnp.int32, (T, T), 0)
    cols = lax.broadcasted_iota(jnp.int32, (T, T), 1)
    attn = attn + jnp.where(rows == cols, 1.0, 0.0)
    out_ref[...] = jnp.where(cols <= rows, attn, 0.0)


def _attn_matrix(pooled, W_out, b_out, Wq, bq, Wk, bk):
    return pl.pallas_call(
        _attn_body,
        out_shape=jax.ShapeDtypeStruct((T, T), jnp.float32),
    )(pooled, W_out, b_out.reshape(1, D), _pe_rows(),
      Wq, bq.reshape(1, D), Wk, bk.reshape(1, D))


def _build_adjacency(edge_index, edge_vals):
    ei = edge_index.reshape(T * 2 * E // SCATTER_W, SCATTER_W)
    ev = edge_vals.reshape(T * E // SCATTER_W, SCATTER_W)
    mesh = plsc.VectorSubcoreMesh(core_axis_name="c", subcore_axis_name="s")

    def body(ei_hbm, ev_hbm, a_hbm, src_v, dst_v,
             idx0_v, idx1_v, val0_v, val1_v, zeros_v,
             sem0, sem1, sh0, sh1):
        core = lax.axis_index("c")
        sid = lax.axis_index("s")

        @pl.loop(0, ZCHUNK, step=16)
        def _(i):
            zeros_v[pl.ds(i, 16)] = jnp.zeros((16,), jnp.float32)

        my_slice = sid * SLICE
        for sh in (sh0, sh1):
            pltpu.sync_copy(zeros_v, sh.at[pl.ds(my_slice, ZCHUNK)])
            pltpu.sync_copy(zeros_v, sh.at[pl.ds(my_slice + ZCHUNK, ZCHUNK)])

        @pl.when(sid == 0)
        def _():
            pltpu.sync_copy(zeros_v.at[pl.ds(0, DUMP)],
                            sh0.at[pl.ds(WIN, DUMP)])
            pltpu.sync_copy(zeros_v.at[pl.ds(0, DUMP)],
                            sh1.at[pl.ds(WIN, DUMP)])
        plsc.subcore_barrier()

        bufs = ((sh0, idx0_v, val0_v, sem0), (sh1, idx1_v, val1_v, sem1))
        pending = [None, None]
        for ps in range(N_PASSES):
            sh, idx_v, val_v, sem = bufs[ps % 2]
            ss, w = ps // 2, ps % 2
            s = core * SNAPS_PER_CORE + ss
            if pending[ps % 2] is not None:
                pending[ps % 2].wait()
                plsc.subcore_barrier()
                for j in range(SCATTER_ROWS):
                    @pl.loop(0, SCATTER_W, step=16)
                    def _(i, j=j):
                        sl = pl.ds(i, 16)
                        val_v[j, sl] = -val_v[j, sl]
                for j in range(SCATTER_ROWS):
                    pltpu.sync_copy(val_v.at[j], sh.at[idx_v.at[j]],
                                    add=True)
            r_src = pl.multiple_of(
                s * 2 * (E // SCATTER_W) + sid * SCATTER_ROWS, 8)
            r_dst = pl.multiple_of(
                (s * 2 + 1) * (E // SCATTER_W) + sid * SCATTER_ROWS, 8)
            r_val = pl.multiple_of(
                s * (E // SCATTER_W) + sid * SCATTER_ROWS, 8)
            pltpu.sync_copy(ei_hbm.at[pl.ds(r_src, SCATTER_ROWS)], src_v)
            pltpu.sync_copy(ei_hbm.at[pl.ds(r_dst, SCATTER_ROWS)], dst_v)
            pltpu.sync_copy(ev_hbm.at[pl.ds(r_val, SCATTER_ROWS)], val_v)
            for j in range(SCATTER_ROWS):
                @pl.loop(0, SCATTER_W, step=16)
                def _(i, j=j):
                    sl = pl.ds(i, 16)
                    ix = src_v[j, sl] * N_NODES + dst_v[j, sl] - w * WIN
                    inr = (ix >= 0) & (ix < WIN)
                    idx_v[j, sl] = jnp.where(
                        inr, ix, WIN + (dst_v[j, sl] & (DUMP - 1)))
            for j in range(SCATTER_ROWS):
                pltpu.sync_copy(val_v.at[j], sh.at[idx_v.at[j]],
                                add=True)
            plsc.subcore_barrier()
            out0 = s * NN + w * WIN + my_slice
            pending[ps % 2] = pltpu.async_copy(
                sh.at[pl.ds(my_slice, SLICE)],
                a_hbm.at[pl.ds(out0, SLICE)], sem)
        pending[0].wait()
        pending[1].wait()

    kern = pl.kernel(
        body,
        out_type=jax.ShapeDtypeStruct((T * NN,), jnp.float32),
        mesh=mesh,
        scratch_types=[
            pltpu.VMEM((SCATTER_ROWS, SCATTER_W), jnp.int32),
            pltpu.VMEM((SCATTER_ROWS, SCATTER_W), jnp.int32),
            pltpu.VMEM((SCATTER_ROWS, SCATTER_W), jnp.int32),
            pltpu.VMEM((SCATTER_ROWS, SCATTER_W), jnp.int32),
            pltpu.VMEM((SCATTER_ROWS, SCATTER_W), jnp.float32),
            pltpu.VMEM((SCATTER_ROWS, SCATTER_W), jnp.float32),
            pltpu.VMEM((ZCHUNK,), jnp.float32),
            pltpu.SemaphoreType.DMA,
            pltpu.SemaphoreType.DMA,
            pltpu.VMEM_SHARED((WIN + DUMP,), jnp.float32),
            pltpu.VMEM_SHARED((WIN + DUMP,), jnp.float32),
        ],
    )
    return kern(ei, ev)


MIX_ROWS = 64


def _mix_body(attn_ref, a_ref, out_ref):
    attn = attn_ref[...]
    for r in range(MIX_ROWS):
        out_ref[:, r, :] = jnp.dot(attn, a_ref[:, r, :],
                                   preferred_element_type=jnp.float32)


def _mix(attn, a_stack):
    return pl.pallas_call(
        _mix_body,
        grid=(N_NODES // MIX_ROWS,),
        in_specs=[
            pl.BlockSpec((T, T), lambda j: (0, 0)),
            pl.BlockSpec((T, MIX_ROWS, N_NODES), lambda j: (0, j, 0)),
        ],
        out_specs=pl.BlockSpec((T, MIX_ROWS, N_NODES), lambda j: (0, j, 0)),
        out_shape=jax.ShapeDtypeStruct((T, N_NODES, N_NODES), jnp.float32),
    )(attn, a_stack)


def kernel(embeddings, edge_vals, W_in, b_in, W_out, b_out, Wq, bq, Wk, bk,
           edge_index):
    pooled = _pooled(embeddings, W_in, b_in)
    attn = _attn_matrix(pooled, W_out, b_out, Wq, bq, Wk, bk)
    a_stack = _build_adjacency(edge_index, edge_vals).reshape(
        T, N_NODES, N_NODES)
    return _mix(attn, a_stack)


# speedup vs baseline: 9.0793x; 1.0284x over previous
import math

import numpy as np
import jax
import jax.numpy as jnp
from jax import lax
from jax.experimental import pallas as pl
from jax.experimental.pallas import tpu as pltpu
from jax.experimental.pallas import tpu_sc as plsc

T = 16
N_ACTIVE = 1024
D = 512
H = 8
N_NODES = 1024
E = 16384

NN = N_NODES * N_NODES
N_SUBCORES = 16
N_SC_CORES = 2
EDGES_PER_SUBCORE = E // N_SUBCORES
SCATTER_W = 128
SCATTER_ROWS = EDGES_PER_SUBCORE // SCATTER_W
SNAPS_PER_CORE = T // N_SC_CORES
WIN = NN // 2
SLICE = WIN // N_SUBCORES
ZCHUNK = SLICE // 2
DUMP = 128
N_PASSES = SNAPS_PER_CORE * 2


def _pe_rows():
    pe = np.zeros((T, D), dtype=np.float32)
    position = np.arange(0, T, dtype=np.float32)[:, None]
    div_term = np.exp(
        np.arange(0, D, 2, dtype=np.float32) * -(math.log(10000.0) / D))
    pe[:, 0::2] = np.sin(position * div_term)
    pe[:, 1::2] = np.cos(position * div_term)
    return jnp.asarray(pe)


def _pool_body(emb_ref, w_ref, b_ref, out_ref):
    em = jnp.dot(emb_ref[0], w_ref[...], preferred_element_type=jnp.float32)
    em = jnp.maximum(em + b_ref[...], 0.0)
    out_ref[0, 0, :] = jnp.mean(em, axis=0) + jnp.max(em, axis=0)


def _pooled(embeddings, W_in, b_in):
    return pl.pallas_call(
        _pool_body,
        grid=(T,),
        in_specs=[
            pl.BlockSpec((1, N_ACTIVE, D), lambda t: (t, 0, 0)),
            pl.BlockSpec((D, 2 * D), lambda t: (0, 0)),
            pl.BlockSpec((1, 2 * D), lambda t: (0, 0)),
        ],
        out_specs=pl.BlockSpec((1, 1, 2 * D), lambda t: (t, 0, 0)),
        out_shape=jax.ShapeDtypeStruct((T, 1, 2 * D), jnp.float32),
    )(embeddings, W_in, b_in.reshape(1, 2 * D)).reshape(T, 2 * D)


def _attn_body(pooled_ref, wo_ref, bo_ref, pe_ref, wq_ref, bq_ref,
               wk_ref, bk_ref, out_ref):
    x = jnp.dot(pooled_ref[...], wo_ref[...],
                preferred_element_type=jnp.float32) + bo_ref[...]
    x = jnp.maximum(x, 0.0) + pe_ref[...]
    head_dim = D // H
    scaling = head_dim ** -0.5
    q = (jnp.dot(x, wq_ref[...], preferred_element_type=jnp.float32)
         + bq_ref[...]) * scaling
    k = jnp.dot(x, wk_ref[...], preferred_element_type=jnp.float32) + bk_ref[...]
    acc = jnp.zeros((T, T), jnp.float32)
    for h in range(H):
        qh = q[:, h * head_dim:(h + 1) * head_dim]
        kh = k[:, h * head_dim:(h + 1) * head_dim]
        logits = lax.dot_general(qh, kh, (((1,), (1,)), ((), ())),
                                 preferred_element_type=jnp.float32)
        m = jnp.max(logits, axis=-1, keepdims=True)
        e = jnp.exp(logits - m)
        acc = acc + e / jnp.sum(e, axis=-1, keepdims=True)
    attn = acc / H
    attn = jnp.where(attn < 1.0 / T, 0.0, attn)
    rows = lax.broadcasted_iota(jnp.int32, (T, T), 0)
    cols = lax.broadcasted_iota(jnp.int32, (T, T), 1)
    attn = attn + jnp.where(rows == cols, 1.0, 0.0)
    out_ref[...] = jnp.where(cols <= rows, attn, 0.0)


def _attn_matrix(pooled, W_out, b_out, Wq, bq, Wk, bk):
    return pl.pallas_call(
        _attn_body,
        out_shape=jax.ShapeDtypeStruct((T, T), jnp.float32),
    )(pooled, W_out, b_out.reshape(1, D), _pe_rows(),
      Wq, bq.reshape(1, D), Wk, bk.reshape(1, D))


def _build_adjacency(edge_index, edge_vals):
    ei = edge_index.reshape(T * 2 * E // SCATTER_W, SCATTER_W)
    ev = edge_vals.reshape(T * E // SCATTER_W, SCATTER_W)
    mesh = plsc.VectorSubcoreMesh(core_axis_name="c", subcore_axis_name="s")

    def body(ei_hbm, ev_hbm, a_hbm, src_v, dst_v,
             idx0_v, idx1_v, val0_v, val1_v, zeros_v,
             sem0, sem1, sh0, sh1):
        core = lax.axis_index("c")
        sid = lax.axis_index("s")

        @pl.loop(0, ZCHUNK, step=16)
        def _(i):
            zeros_v[pl.ds(i, 16)] = jnp.zeros((16,), jnp.float32)

        my_slice = sid * SLICE
        for sh in (sh0, sh1):
            pltpu.sync_copy(zeros_v, sh.at[pl.ds(my_slice, ZCHUNK)])
            pltpu.sync_copy(zeros_v, sh.at[pl.ds(my_slice + ZCHUNK, ZCHUNK)])

        @pl.when(sid == 0)
        def _():
            pltpu.sync_copy(zeros_v.at[pl.ds(0, DUMP)],
                            sh0.at[pl.ds(WIN, DUMP)])
            pltpu.sync_copy(zeros_v.at[pl.ds(0, DUMP)],
                            sh1.at[pl.ds(WIN, DUMP)])
        plsc.subcore_barrier()

        bufs = ((sh0, idx0_v, val0_v, sem0), (sh1, idx1_v, val1_v, sem1))
        pending = [None, None]
        for ps in range(N_PASSES):
            sh, idx_v, val_v, sem = bufs[ps % 2]
            ss, w = ps // 2, ps % 2
            s = core * SNAPS_PER_CORE + ss
            if pending[ps % 2] is not None:
                pending[ps % 2].wait()
                plsc.subcore_barrier()
                for j in range(SCATTER_ROWS):
                    @pl.loop(0, SCATTER_W, step=16)
                    def _(i, j=j):
                        sl = pl.ds(i, 16)
                        val_v[j, sl] = -val_v[j, sl]
                for j in range(SCATTER_ROWS):
                    pltpu.sync_copy(val_v.at[j], sh.at[idx_v.at[j]],
                                    add=True)
            r_src = pl.multiple_of(
                s * 2 * (E // SCATTER_W) + sid * SCATTER_ROWS, 8)
            r_dst = pl.multiple_of(
                (s * 2 + 1) * (E // SCATTER_W) + sid * SCATTER_ROWS, 8)
            r_val = pl.multiple_of(
                s * (E // SCATTER_W) + sid * SCATTER_ROWS, 8)
            pltpu.sync_copy(ei_hbm.at[pl.ds(r_src, SCATTER_ROWS)], src_v)
            pltpu.sync_copy(ei_hbm.at[pl.ds(r_dst, SCATTER_ROWS)], dst_v)
            pltpu.sync_copy(ev_hbm.at[pl.ds(r_val, SCATTER_ROWS)], val_v)
            for j in range(SCATTER_ROWS):
                @pl.loop(0, SCATTER_W, step=16)
                def _(i, j=j):
                    sl = pl.ds(i, 16)
                    ix = src_v[j, sl] * N_NODES + dst_v[j, sl] - w * WIN
                    inr = (ix >= 0) & (ix < WIN)
                    idx_v[j, sl] = jnp.where(
                        inr, ix, WIN + (dst_v[j, sl] & (DUMP - 1)))
            for j in range(SCATTER_ROWS):
                pltpu.sync_copy(val_v.at[j], sh.at[idx_v.at[j]],
                                add=True)
            plsc.subcore_barrier()
            out0 = s * NN + w * WIN + my_slice
            pending[ps % 2] = pltpu.async_copy(
                sh.at[pl.ds(my_slice, SLICE)],
                a_hbm.at[pl.ds(out0, SLICE)], sem)
        pending[0].wait()
        pending[1].wait()

    kern = pl.kernel(
        body,
        out_type=jax.ShapeDtypeStruct((T * NN,), jnp.float32),
        mesh=mesh,
        scratch_types=[
            pltpu.VMEM((SCATTER_ROWS, SCATTER_W), jnp.int32),
            pltpu.VMEM((SCATTER_ROWS, SCATTER_W), jnp.int32),
            pltpu.VMEM((SCATTER_ROWS, SCATTER_W), jnp.int32),
            pltpu.VMEM((SCATTER_ROWS, SCATTER_W), jnp.int32),
            pltpu.VMEM((SCATTER_ROWS, SCATTER_W), jnp.float32),
            pltpu.VMEM((SCATTER_ROWS, SCATTER_W), jnp.float32),
            pltpu.VMEM((ZCHUNK,), jnp.float32),
            pltpu.SemaphoreType.DMA,
            pltpu.SemaphoreType.DMA,
            pltpu.VMEM_SHARED((WIN + DUMP,), jnp.float32),
            pltpu.VMEM_SHARED((WIN + DUMP,), jnp.float32),
        ],
    )
    return kern(ei, ev)


MIX_ROWS = 128


def _mix_body(attn_ref, a_ref, out_ref):
    out_ref[...] = lax.dot_general(
        attn_ref[...], a_ref[...], (((1,), (0,)), ((), ())),
        preferred_element_type=jnp.float32)


def _mix(attn, a_stack):
    return pl.pallas_call(
        _mix_body,
        grid=(N_NODES // MIX_ROWS,),
        in_specs=[
            pl.BlockSpec((T, T), lambda j: (0, 0)),
            pl.BlockSpec((T, MIX_ROWS, N_NODES), lambda j: (0, j, 0)),
        ],
        out_specs=pl.BlockSpec((T, MIX_ROWS, N_NODES), lambda j: (0, j, 0)),
        out_shape=jax.ShapeDtypeStruct((T, N_NODES, N_NODES), jnp.float32),
    )(attn, a_stack)


def kernel(embeddings, edge_vals, W_in, b_in, W_out, b_out, Wq, bq, Wk, bk,
           edge_index):
    pooled = _pooled(embeddings, W_in, b_in)
    attn = _attn_matrix(pooled, W_out, b_out, Wq, bq, Wk, bk)
    a_stack = _build_adjacency(edge_index, edge_vals).reshape(
        T, N_NODES, N_NODES)
    return _mix(attn, a_stack)
